# Initial kernel scaffold; baseline (speedup 1.0000x reference)
import jax
import jax.numpy as jnp
from jax.experimental import pallas as pl


def kernel(h, edges_a, edges_b, update_mask_a, update_mask_b, emb_in_w, emb_in_b, emb_out_w, emb_out_b, ew1, eb1, ew2, eb2, nw1, nb1, nw2, nb2):
    raise NotImplementedError("write your pallas kernel here")


# speedup vs baseline: 2.5781x; 2.5781x over previous
import functools

import jax
import jax.numpy as jnp
from jax import lax
from jax.experimental import pallas as pl
from jax.experimental.pallas import tpu as pltpu
from jax.experimental.pallas import tpu_sc as plsc

N = 10000
E = 320000
D = 128
EH = 64
L = 4
NUM_LATENT = 1000

NC = 2
NS = 16
NW = NC * NS
EPW = E // NW
K = 80
NCHUNK = EPW // K
EPS = E // NS
SCHUNK = EPS // K
NHALF = N // 2
ZRS = 320
NACC = NS * ZRS
TRASH = 5056
W = 128

BN = 2000
BE = 4000

f32 = jnp.float32
_mesh = plsc.VectorSubcoreMesh(core_axis_name="c", subcore_axis_name="s")


def _silu(x):
    return x * jax.nn.sigmoid(x)


@functools.partial(
    pl.kernel,
    mesh=_mesh,
    out_type=jax.ShapeDtypeStruct((E, EH), f32),
    scratch_types=[
        pltpu.VMEM((NCHUNK, K), jnp.int32),
        pltpu.VMEM((NCHUNK, K), jnp.int32),
        pltpu.VMEM((K, D), f32),
        pltpu.VMEM((K, D), f32),
        pltpu.VMEM((K, EH), f32),
        pltpu.SemaphoreType.DMA,
        pltpu.SemaphoreType.DMA,
    ],
)
def _sc_gather_add(pq_hbm, rows_hbm, cols_hbm, s_hbm,
                   idxr, idxc, bufr, bufc, sbuf, sem1, sem2):
    wid = lax.axis_index("s") * NC + lax.axis_index("c")
    pltpu.sync_copy(rows_hbm.at[wid], idxr)
    pltpu.sync_copy(cols_hbm.at[wid], idxc)

    def chunk(j, carry):
        cp = pltpu.async_copy(pq_hbm.at[idxr.at[j]], bufr, sem1)
        cq = pltpu.async_copy(pq_hbm.at[idxc.at[j]], bufc, sem2)
        cp.wait()
        cq.wait()

        def addrow(r, c2):
            for t in range(EH // 16):
                sbuf[r, pl.ds(t * 16, 16)] = (bufr[r, pl.ds(t * 16, 16)]
                                              + bufc[r, pl.ds(EH + t * 16, 16)])
            return c2

        lax.fori_loop(0, K, addrow, 0)
        pltpu.sync_copy(sbuf, s_hbm.at[pl.ds(wid * EPW + j * K, K)])
        return carry

    lax.fori_loop(0, NCHUNK, chunk, 0)


@functools.partial(
    pl.kernel,
    mesh=_mesh,
    out_type=jax.ShapeDtypeStruct((N, EH), f32),
    scratch_types=[
        pltpu.VMEM((K,), jnp.int32),
        pltpu.VMEM((K,), jnp.int32),
        pltpu.VMEM((K, W), f32),
        pltpu.VMEM((K, EH), f32),
        pltpu.VMEM_SHARED((NACC, W), f32),
    ],
)
def _sc_scatter_add(m_hbm, rows_hbm, out_hbm, idxc, tidx, bufm, pbuf, acc):
    cid = lax.axis_index("c")
    sid = lax.axis_index("s")
    zero = jnp.zeros((16,), f32)

    def zrow(r, c2):
        for t in range(W // 16):
            bufm[r, pl.ds(t * 16, 16)] = zero
        return c2

    lax.fori_loop(0, K, zrow, 0)
    for g in range(ZRS // K):
        pltpu.sync_copy(bufm, acc.at[pl.ds(sid * ZRS + g * K, K)])
    plsc.subcore_barrier()

    base = cid * NHALF

    def chunk(j, carry):
        pltpu.sync_copy(rows_hbm.at[sid, j], idxc)
        pltpu.sync_copy(m_hbm.at[pl.ds(sid * EPS + j * K, K)], bufm)
        for v in range(K // 16):
            sl = pl.ds(v * 16, 16)
            loc = idxc[sl] - base
            ok = (loc >= 0) & (loc < NHALF)
            tidx[sl] = jnp.where(ok, loc, TRASH)
        pltpu.sync_copy(bufm, acc.at[tidx], add=True)
        return carry

    lax.fori_loop(0, SCHUNK, chunk, 0)
    plsc.subcore_barrier()

    def pack_group(g_rows, acc_off, out_off):
        pltpu.sync_copy(acc.at[pl.ds(acc_off, K)], bufm)

        def prow(r, c2):
            for t in range(EH // 16):
                sl = pl.ds(t * 16, 16)
                pbuf[r, sl] = bufm[r, sl]
            return c2

        lax.fori_loop(0, g_rows, prow, 0)
        pltpu.sync_copy(pbuf.at[pl.ds(0, g_rows)],
                        out_hbm.at[pl.ds(out_off, g_rows)])

    @pl.when(sid < NS - 1)
    def _copy_full():
        for g in range(ZRS // K):
            pack_group(K, sid * ZRS + g * K, base + sid * ZRS + g * K)

    @pl.when(sid == NS - 1)
    def _copy_tail():
        rem = NHALF - (NS - 1) * ZRS
        for g in range(rem // 40):
            pack_group(40, (NS - 1) * ZRS + g * 40,
                       base + (NS - 1) * ZRS + g * 40)


def _embed_pq_body(h_ref, we_ref, be_ref, w1_ref, b1_ref, h0_ref, pq_ref):
    h0 = jnp.dot(h_ref[...], we_ref[...], preferred_element_type=f32) + be_ref[...]
    h0_ref[...] = h0
    pq_ref[...] = jnp.dot(h0, w1_ref[...], preferred_element_type=f32) + b1_ref[...]


def _tc_embed_pq(h, we, be, w1, b1):
    return pl.pallas_call(
        _embed_pq_body,
        grid=(N // BN,),
        in_specs=[
            pl.BlockSpec((BN, D), lambda i: (i, 0)),
            pl.BlockSpec((D, D), lambda i: (0, 0)),
            pl.BlockSpec((1, D), lambda i: (0, 0)),
            pl.BlockSpec((D, 2 * EH), lambda i: (0, 0)),
            pl.BlockSpec((1, 2 * EH), lambda i: (0, 0)),
        ],
        out_specs=[
            pl.BlockSpec((BN, D), lambda i: (i, 0)),
            pl.BlockSpec((BN, 2 * EH), lambda i: (i, 0)),
        ],
        out_shape=[
            jax.ShapeDtypeStruct((N, D), f32),
            jax.ShapeDtypeStruct((N, 2 * EH), f32),
        ],
    )(h, we, be, w1, b1)


def _edge_mlp_body(s_ref, w2_ref, b2_ref, m_ref, *, c):
    s = _silu(s_ref[...])
    t = jnp.dot(s, w2_ref[...], preferred_element_type=f32) + b2_ref[...]
    m_ref[...] = jnp.concatenate(
        [_silu(t) * c, jnp.zeros((t.shape[0], W - EH), f32)], axis=1)


def _tc_edge_mlp(s, w2, b2, c):
    return pl.pallas_call(
        functools.partial(_edge_mlp_body, c=c),
        grid=(E // BE,),
        in_specs=[
            pl.BlockSpec((BE, EH), lambda i: (i, 0)),
            pl.BlockSpec((EH, EH), lambda i: (0, 0)),
            pl.BlockSpec((1, EH), lambda i: (0, 0)),
        ],
        out_specs=pl.BlockSpec((BE, W), lambda i: (i, 0)),
        out_shape=jax.ShapeDtypeStruct((E, W), f32),
    )(s, w2, b2)


def _node_body(h_ref, agg_ref, mask_ref, nh_ref, na_ref, nb1_ref, w2_ref,
               nb2_ref, wx_ref, bx_ref, hn_ref, pq_ref):
    h0 = h_ref[...]
    agg = agg_ref[...]
    t = (jnp.dot(h0, nh_ref[...], preferred_element_type=f32)
         + jnp.dot(agg, na_ref[...], preferred_element_type=f32) + nb1_ref[...])
    t = _silu(t)
    out = jnp.dot(t, w2_ref[...], preferred_element_type=f32) + nb2_ref[...]
    hn = h0 + mask_ref[...] * out
    hn_ref[...] = hn
    pq_ref[...] = jnp.dot(hn, wx_ref[...], preferred_element_type=f32) + bx_ref[...]


def _tc_node(h0, agg, mask, nh, na, nb1, nw2i, nb2, wx, bx):
    return pl.pallas_call(
        _node_body,
        grid=(N // BN,),
        in_specs=[
            pl.BlockSpec((BN, D), lambda i: (i, 0)),
            pl.BlockSpec((BN, EH), lambda i: (i, 0)),
            pl.BlockSpec((BN, 1), lambda i: (i, 0)),
            pl.BlockSpec((D, D), lambda i: (0, 0)),
            pl.BlockSpec((EH, D), lambda i: (0, 0)),
            pl.BlockSpec((1, D), lambda i: (0, 0)),
            pl.BlockSpec((D, D), lambda i: (0, 0)),
            pl.BlockSpec((1, D), lambda i: (0, 0)),
            pl.BlockSpec((D, 2 * EH), lambda i: (0, 0)),
            pl.BlockSpec((1, 2 * EH), lambda i: (0, 0)),
        ],
        out_specs=[
            pl.BlockSpec((BN, D), lambda i: (i, 0)),
            pl.BlockSpec((BN, 2 * EH), lambda i: (i, 0)),
        ],
        out_shape=[
            jax.ShapeDtypeStruct((N, D), f32),
            jax.ShapeDtypeStruct((N, 2 * EH), f32),
        ],
    )(h0, agg, mask, nh, na, nb1, nw2i, nb2, wx, bx)


def _node_final_body(h_ref, agg_ref, mask_ref, nh_ref, na_ref, nb1_ref,
                     w2_ref, nb2_ref, wo_ref, bo_ref, out_ref):
    h0 = h_ref[...]
    agg = agg_ref[...]
    t = (jnp.dot(h0, nh_ref[...], preferred_element_type=f32)
         + jnp.dot(agg, na_ref[...], preferred_element_type=f32) + nb1_ref[...])
    t = _silu(t)
    out = jnp.dot(t, w2_ref[...], preferred_element_type=f32) + nb2_ref[...]
    hn = h0 + mask_ref[...] * out
    out_ref[...] = jnp.dot(hn, wo_ref[...], preferred_element_type=f32) + bo_ref[...]


def _tc_node_final(h0, agg, mask, nh, na, nb1, nw2i, nb2, wo, bo):
    return pl.pallas_call(
        _node_final_body,
        grid=(N // BN,),
        in_specs=[
            pl.BlockSpec((BN, D), lambda i: (i, 0)),
            pl.BlockSpec((BN, EH), lambda i: (i, 0)),
            pl.BlockSpec((BN, 1), lambda i: (i, 0)),
            pl.BlockSpec((D, D), lambda i: (0, 0)),
            pl.BlockSpec((EH, D), lambda i: (0, 0)),
            pl.BlockSpec((1, D), lambda i: (0, 0)),
            pl.BlockSpec((D, D), lambda i: (0, 0)),
            pl.BlockSpec((1, D), lambda i: (0, 0)),
            pl.BlockSpec((D, D), lambda i: (0, 0)),
            pl.BlockSpec((1, D), lambda i: (0, 0)),
        ],
        out_specs=pl.BlockSpec((BN, D), lambda i: (i, 0)),
        out_shape=jax.ShapeDtypeStruct((N, D), f32),
    )(h0, agg, mask, nh, na, nb1, nw2i, nb2, wo, bo)


def kernel(h, edges_a, edges_b, update_mask_a, update_mask_b,
           emb_in_w, emb_in_b, emb_out_w, emb_out_b,
           ew1, eb1, ew2, eb2, nw1, nb1, nw2, nb2):
    rows_a = edges_a[0].reshape(NW, NCHUNK, K)
    cols_a = edges_a[1].reshape(NW, NCHUNK, K)
    rows_b = edges_b[0].reshape(NW, NCHUNK, K)
    cols_b = edges_b[1].reshape(NW, NCHUNK, K)
    srows_a = edges_a[0].reshape(NS, SCHUNK, K)
    srows_b = edges_b[0].reshape(NS, SCHUNK, K)
    w1p = jnp.concatenate([ew1[:, :D, :], ew1[:, D:, :]], axis=-1)
    b1p = jnp.concatenate([eb1, jnp.zeros_like(eb1)], axis=-1)
    nh = nw1[:, :D, :]
    na = nw1[:, D:, :]

    h0, pq = _tc_embed_pq(h, emb_in_w, emb_in_b.reshape(1, D),
                          w1p[0], b1p[0].reshape(1, 2 * EH))
    out = None
    for i in range(L):
        rows, cols = (rows_a, cols_a) if i % 2 == 0 else (rows_b, cols_b)
        srows = srows_a if i % 2 == 0 else srows_b
        mask = update_mask_a if i % 2 == 0 else update_mask_b
        c = 1.0 if i % 2 == 0 else 2.0 / NUM_LATENT
        s = _sc_gather_add(pq, rows, cols)
        m = _tc_edge_mlp(s, ew2[i], eb2[i].reshape(1, EH), c)
        agg = _sc_scatter_add(m, srows)
        if i < L - 1:
            h0, pq = _tc_node(h0, agg, mask, nh[i], na[i],
                              nb1[i].reshape(1, D), nw2[i],
                              nb2[i].reshape(1, D), w1p[i + 1],
                              b1p[i + 1].reshape(1, 2 * EH))
        else:
            out = _tc_node_final(h0, agg, mask, nh[i], na[i],
                                 nb1[i].reshape(1, D), nw2[i],
                                 nb2[i].reshape(1, D), emb_out_w,
                                 emb_out_b.reshape(1, D))
    return out


# speedup vs baseline: 3.7612x; 1.4589x over previous
import functools

import jax
import jax.numpy as jnp
from jax import lax
from jax.experimental import pallas as pl
from jax.experimental.pallas import tpu as pltpu
from jax.experimental.pallas import tpu_sc as plsc

N = 10000
E = 320000
D = 128
EH = 64
L = 4
NUM_LATENT = 1000

NC = 2
NS = 16
NW = NC * NS
EPW = E // NW
K = 80
NCHUNK = EPW // K
EPS = E // NS
SCHUNK = EPS // K
NHALF = N // 2
ZRS = 320
NACC = NS * ZRS
TRASH = 5056
W = 128

BN = 2000
BE = 4000

f32 = jnp.float32
_mesh = plsc.VectorSubcoreMesh(core_axis_name="c", subcore_axis_name="s")


def _silu(x):
    return x * jax.nn.sigmoid(x)


@functools.partial(
    pl.kernel,
    mesh=_mesh,
    out_type=jax.ShapeDtypeStruct((E, EH), f32),
    scratch_types=[
        pltpu.VMEM((NCHUNK, K), jnp.int32),
        pltpu.VMEM((NCHUNK, K), jnp.int32),
        pltpu.VMEM((K, D), f32),
        pltpu.VMEM((K, D), f32),
        pltpu.VMEM((K, D), f32),
        pltpu.VMEM((K, D), f32),
        pltpu.VMEM((K, EH), f32),
        pltpu.VMEM((K, EH), f32),
        pltpu.SemaphoreType.DMA,
        pltpu.SemaphoreType.DMA,
        pltpu.SemaphoreType.DMA,
        pltpu.SemaphoreType.DMA,
        pltpu.SemaphoreType.DMA,
        pltpu.SemaphoreType.DMA,
    ],
)
def _sc_gather_add(pq_hbm, rows_hbm, cols_hbm, s_hbm,
                   idxr, idxc, bufr0, bufc0, bufr1, bufc1, sbuf0, sbuf1,
                   semr0, semc0, semr1, semc1, sems0, sems1):
    wid = lax.axis_index("s") * NC + lax.axis_index("c")
    pltpu.sync_copy(rows_hbm.at[wid], idxr)
    pltpu.sync_copy(cols_hbm.at[wid], idxc)

    slots = ((bufr0, bufc0, sbuf0, semr0, semc0, sems0),
             (bufr1, bufc1, sbuf1, semr1, semc1, sems1))

    def issue(j, s):
        bufr, bufc, _, semr, semc, _ = slots[s]
        pltpu.async_copy(pq_hbm.at[idxr.at[j]], bufr, semr)
        pltpu.async_copy(pq_hbm.at[idxc.at[j]], bufc, semc)

    def process(j, s, first):
        bufr, bufc, sbuf, semr, semc, sems = slots[s]
        pltpu.make_async_copy(pq_hbm.at[idxr.at[j]], bufr, semr).wait()
        pltpu.make_async_copy(pq_hbm.at[idxc.at[j]], bufc, semc).wait()
        if not first:
            pltpu.make_async_copy(
                sbuf, s_hbm.at[pl.ds(wid * EPW, K)], sems).wait()

        def addrow(r, c2):
            for t in range(EH // 16):
                sbuf[r, pl.ds(t * 16, 16)] = (bufr[r, pl.ds(t * 16, 16)]
                                              + bufc[r, pl.ds(EH + t * 16, 16)])
            return c2

        lax.fori_loop(0, K, addrow, 0)
        pltpu.async_copy(sbuf, s_hbm.at[pl.ds(wid * EPW + j * K, K)], sems)

    issue(0, 0)
    issue(1, 1)

    def pair(it, carry):
        j0 = it * 2

        @pl.when(it > 0)
        def _p0():
            process(j0, 0, False)

        @pl.when(it == 0)
        def _p0f():
            process(j0, 0, True)

        @pl.when(j0 + 2 < NCHUNK)
        def _i0():
            issue(j0 + 2, 0)

        @pl.when(it > 0)
        def _p1():
            process(j0 + 1, 1, False)

        @pl.when(it == 0)
        def _p1f():
            process(j0 + 1, 1, True)

        @pl.when(j0 + 3 < NCHUNK)
        def _i1():
            issue(j0 + 3, 1)

        return carry

    lax.fori_loop(0, NCHUNK // 2, pair, 0)
    process(NCHUNK - 1, 0, False)
    pltpu.make_async_copy(sbuf1, s_hbm.at[pl.ds(wid * EPW, K)], sems1).wait()
    pltpu.make_async_copy(sbuf0, s_hbm.at[pl.ds(wid * EPW, K)], sems0).wait()


@functools.partial(
    pl.kernel,
    mesh=_mesh,
    out_type=jax.ShapeDtypeStruct((N, EH), f32),
    scratch_types=[
        pltpu.VMEM((K,), jnp.int32),
        pltpu.VMEM((K,), jnp.int32),
        pltpu.VMEM((K,), jnp.int32),
        pltpu.VMEM((K,), jnp.int32),
        pltpu.VMEM((K, W), f32),
        pltpu.VMEM((K, W), f32),
        pltpu.VMEM((K, EH), f32),
        pltpu.VMEM_SHARED((NACC, W), f32),
        pltpu.SemaphoreType.DMA,
        pltpu.SemaphoreType.DMA,
        pltpu.SemaphoreType.DMA,
        pltpu.SemaphoreType.DMA,
        pltpu.SemaphoreType.DMA,
        pltpu.SemaphoreType.DMA,
    ],
)
def _sc_scatter_add(m_hbm, rows_hbm, out_hbm, idxc0, idxc1, tidx0, tidx1,
                    bufm0, bufm1, pbuf, acc,
                    semi0, semi1, semm0, semm1, semsc0, semsc1):
    cid = lax.axis_index("c")
    sid = lax.axis_index("s")
    zero = jnp.zeros((16,), f32)

    def zrow(r, c2):
        for t in range(W // 16):
            bufm0[r, pl.ds(t * 16, 16)] = zero
        return c2

    lax.fori_loop(0, K, zrow, 0)
    for g in range(ZRS // K):
        pltpu.sync_copy(bufm0, acc.at[pl.ds(sid * ZRS + g * K, K)])
    plsc.subcore_barrier()

    base = cid * NHALF
    slots = ((idxc0, tidx0, bufm0, semi0, semm0, semsc0),
             (idxc1, tidx1, bufm1, semi1, semm1, semsc1))

    def issue(j, s):
        idxc, _, bufm, semi, semm, _ = slots[s]
        pltpu.async_copy(rows_hbm.at[sid, j], idxc, semi)
        pltpu.async_copy(m_hbm.at[pl.ds(sid * EPS + j * K, K)], bufm, semm)

    def start_scatter(j, s):
        idxc, tidx, bufm, semi, semm, semsc = slots[s]
        pltpu.make_async_copy(rows_hbm.at[sid, j], idxc, semi).wait()
        pltpu.make_async_copy(
            m_hbm.at[pl.ds(sid * EPS + j * K, K)], bufm, semm).wait()
        for v in range(K // 16):
            sl = pl.ds(v * 16, 16)
            loc = idxc[sl] - base
            ok = (loc >= 0) & (loc < NHALF)
            tidx[sl] = jnp.where(ok, loc, TRASH)
        return pltpu.async_copy(bufm, acc.at[tidx], semsc, add=True)

    issue(0, 0)
    issue(1, 1)

    def pair(it, carry):
        j0 = it * 2
        sc0 = start_scatter(j0, 0)
        sc1 = start_scatter(j0 + 1, 1)
        sc0.wait()

        @pl.when(j0 + 2 < SCHUNK)
        def _i0():
            issue(j0 + 2, 0)

        sc1.wait()

        @pl.when(j0 + 3 < SCHUNK)
        def _i1():
            issue(j0 + 3, 1)

        return carry

    lax.fori_loop(0, SCHUNK // 2, pair, 0)
    plsc.subcore_barrier()

    def pack_group(g_rows, acc_off, out_off):
        pltpu.sync_copy(acc.at[pl.ds(acc_off, K)], bufm0)

        def prow(r, c2):
            for t in range(EH // 16):
                sl = pl.ds(t * 16, 16)
                pbuf[r, sl] = bufm0[r, sl]
            return c2

        lax.fori_loop(0, g_rows, prow, 0)
        pltpu.sync_copy(pbuf.at[pl.ds(0, g_rows)],
                        out_hbm.at[pl.ds(out_off, g_rows)])

    @pl.when(sid < NS - 1)
    def _copy_full():
        for g in range(ZRS // K):
            pack_group(K, sid * ZRS + g * K, base + sid * ZRS + g * K)

    @pl.when(sid == NS - 1)
    def _copy_tail():
        rem = NHALF - (NS - 1) * ZRS
        for g in range(rem // 40):
            pack_group(40, (NS - 1) * ZRS + g * 40,
                       base + (NS - 1) * ZRS + g * 40)


def _embed_pq_body(h_ref, we_ref, be_ref, w1_ref, b1_ref, h0_ref, pq_ref):
    h0 = jnp.dot(h_ref[...], we_ref[...], preferred_element_type=f32) + be_ref[...]
    h0_ref[...] = h0
    pq_ref[...] = jnp.dot(h0, w1_ref[...], preferred_element_type=f32) + b1_ref[...]


def _tc_embed_pq(h, we, be, w1, b1):
    return pl.pallas_call(
        _embed_pq_body,
        grid=(N // BN,),
        in_specs=[
            pl.BlockSpec((BN, D), lambda i: (i, 0)),
            pl.BlockSpec((D, D), lambda i: (0, 0)),
            pl.BlockSpec((1, D), lambda i: (0, 0)),
            pl.BlockSpec((D, 2 * EH), lambda i: (0, 0)),
            pl.BlockSpec((1, 2 * EH), lambda i: (0, 0)),
        ],
        out_specs=[
            pl.BlockSpec((BN, D), lambda i: (i, 0)),
            pl.BlockSpec((BN, 2 * EH), lambda i: (i, 0)),
        ],
        out_shape=[
            jax.ShapeDtypeStruct((N, D), f32),
            jax.ShapeDtypeStruct((N, 2 * EH), f32),
        ],
    )(h, we, be, w1, b1)


def _edge_mlp_body(s_ref, w2_ref, b2_ref, m_ref, *, c):
    s = _silu(s_ref[...])
    t = jnp.dot(s, w2_ref[...], preferred_element_type=f32) + b2_ref[...]
    m_ref[...] = jnp.concatenate(
        [_silu(t) * c, jnp.zeros((t.shape[0], W - EH), f32)], axis=1)


def _tc_edge_mlp(s, w2, b2, c):
    return pl.pallas_call(
        functools.partial(_edge_mlp_body, c=c),
        grid=(E // BE,),
        in_specs=[
            pl.BlockSpec((BE, EH), lambda i: (i, 0)),
            pl.BlockSpec((EH, EH), lambda i: (0, 0)),
            pl.BlockSpec((1, EH), lambda i: (0, 0)),
        ],
        out_specs=pl.BlockSpec((BE, W), lambda i: (i, 0)),
        out_shape=jax.ShapeDtypeStruct((E, W), f32),
    )(s, w2, b2)


def _node_body(h_ref, agg_ref, mask_ref, nh_ref, na_ref, nb1_ref, w2_ref,
               nb2_ref, wx_ref, bx_ref, hn_ref, pq_ref):
    h0 = h_ref[...]
    agg = agg_ref[...]
    t = (jnp.dot(h0, nh_ref[...], preferred_element_type=f32)
         + jnp.dot(agg, na_ref[...], preferred_element_type=f32) + nb1_ref[...])
    t = _silu(t)
    out = jnp.dot(t, w2_ref[...], preferred_element_type=f32) + nb2_ref[...]
    hn = h0 + mask_ref[...] * out
    hn_ref[...] = hn
    pq_ref[...] = jnp.dot(hn, wx_ref[...], preferred_element_type=f32) + bx_ref[...]


def _tc_node(h0, agg, mask, nh, na, nb1, nw2i, nb2, wx, bx):
    return pl.pallas_call(
        _node_body,
        grid=(N // BN,),
        in_specs=[
            pl.BlockSpec((BN, D), lambda i: (i, 0)),
            pl.BlockSpec((BN, EH), lambda i: (i, 0)),
            pl.BlockSpec((BN, 1), lambda i: (i, 0)),
            pl.BlockSpec((D, D), lambda i: (0, 0)),
            pl.BlockSpec((EH, D), lambda i: (0, 0)),
            pl.BlockSpec((1, D), lambda i: (0, 0)),
            pl.BlockSpec((D, D), lambda i: (0, 0)),
            pl.BlockSpec((1, D), lambda i: (0, 0)),
            pl.BlockSpec((D, 2 * EH), lambda i: (0, 0)),
            pl.BlockSpec((1, 2 * EH), lambda i: (0, 0)),
        ],
        out_specs=[
            pl.BlockSpec((BN, D), lambda i: (i, 0)),
            pl.BlockSpec((BN, 2 * EH), lambda i: (i, 0)),
        ],
        out_shape=[
            jax.ShapeDtypeStruct((N, D), f32),
            jax.ShapeDtypeStruct((N, 2 * EH), f32),
        ],
    )(h0, agg, mask, nh, na, nb1, nw2i, nb2, wx, bx)


def _node_final_body(h_ref, agg_ref, mask_ref, nh_ref, na_ref, nb1_ref,
                     w2_ref, nb2_ref, wo_ref, bo_ref, out_ref):
    h0 = h_ref[...]
    agg = agg_ref[...]
    t = (jnp.dot(h0, nh_ref[...], preferred_element_type=f32)
         + jnp.dot(agg, na_ref[...], preferred_element_type=f32) + nb1_ref[...])
    t = _silu(t)
    out = jnp.dot(t, w2_ref[...], preferred_element_type=f32) + nb2_ref[...]
    hn = h0 + mask_ref[...] * out
    out_ref[...] = jnp.dot(hn, wo_ref[...], preferred_element_type=f32) + bo_ref[...]


def _tc_node_final(h0, agg, mask, nh, na, nb1, nw2i, nb2, wo, bo):
    return pl.pallas_call(
        _node_final_body,
        grid=(N // BN,),
        in_specs=[
            pl.BlockSpec((BN, D), lambda i: (i, 0)),
            pl.BlockSpec((BN, EH), lambda i: (i, 0)),
            pl.BlockSpec((BN, 1), lambda i: (i, 0)),
            pl.BlockSpec((D, D), lambda i: (0, 0)),
            pl.BlockSpec((EH, D), lambda i: (0, 0)),
            pl.BlockSpec((1, D), lambda i: (0, 0)),
            pl.BlockSpec((D, D), lambda i: (0, 0)),
            pl.BlockSpec((1, D), lambda i: (0, 0)),
            pl.BlockSpec((D, D), lambda i: (0, 0)),
            pl.BlockSpec((1, D), lambda i: (0, 0)),
        ],
        out_specs=pl.BlockSpec((BN, D), lambda i: (i, 0)),
        out_shape=jax.ShapeDtypeStruct((N, D), f32),
    )(h0, agg, mask, nh, na, nb1, nw2i, nb2, wo, bo)


def kernel(h, edges_a, edges_b, update_mask_a, update_mask_b,
           emb_in_w, emb_in_b, emb_out_w, emb_out_b,
           ew1, eb1, ew2, eb2, nw1, nb1, nw2, nb2):
    rows_a = edges_a[0].reshape(NW, NCHUNK, K)
    cols_a = edges_a[1].reshape(NW, NCHUNK, K)
    rows_b = edges_b[0].reshape(NW, NCHUNK, K)
    cols_b = edges_b[1].reshape(NW, NCHUNK, K)
    srows_a = edges_a[0].reshape(NS, SCHUNK, K)
    srows_b = edges_b[0].reshape(NS, SCHUNK, K)
    w1p = jnp.concatenate([ew1[:, :D, :], ew1[:, D:, :]], axis=-1)
    b1p = jnp.concatenate([eb1, jnp.zeros_like(eb1)], axis=-1)
    nh = nw1[:, :D, :]
    na = nw1[:, D:, :]

    h0, pq = _tc_embed_pq(h, emb_in_w, emb_in_b.reshape(1, D),
                          w1p[0], b1p[0].reshape(1, 2 * EH))
    out = None
    for i in range(L):
        rows, cols = (rows_a, cols_a) if i % 2 == 0 else (rows_b, cols_b)
        srows = srows_a if i % 2 == 0 else srows_b
        mask = update_mask_a if i % 2 == 0 else update_mask_b
        c = 1.0 if i % 2 == 0 else 2.0 / NUM_LATENT
        s = _sc_gather_add(pq, rows, cols)
        m = _tc_edge_mlp(s, ew2[i], eb2[i].reshape(1, EH), c)
        agg = _sc_scatter_add(m, srows)
        if i < L - 1:
            h0, pq = _tc_node(h0, agg, mask, nh[i], na[i],
                              nb1[i].reshape(1, D), nw2[i],
                              nb2[i].reshape(1, D), w1p[i + 1],
                              b1p[i + 1].reshape(1, 2 * EH))
        else:
            out = _tc_node_final(h0, agg, mask, nh[i], na[i],
                                 nb1[i].reshape(1, D), nw2[i],
                                 nb2[i].reshape(1, D), emb_out_w,
                                 emb_out_b.reshape(1, D))
    return out


# speedup vs baseline: 4.9173x; 1.3074x over previous
import functools

import jax
import jax.numpy as jnp
from jax import lax
from jax.experimental import pallas as pl
from jax.experimental.pallas import tpu as pltpu
from jax.experimental.pallas import tpu_sc as plsc

N = 10000
E = 320000
D = 128
EH = 64
L = 4
NUM_LATENT = 1000

NC = 2
NS = 16
NW = NC * NS
EPW = E // NW
K = 80
NCHUNK = EPW // K
NACC = 10240
ZRS = NACC // NS
W = 128

BN = 2000
BE = 4000

f32 = jnp.float32
_mesh = plsc.VectorSubcoreMesh(core_axis_name="c", subcore_axis_name="s")


def _silu(x):
    return x * jax.nn.sigmoid(x)


@functools.partial(
    pl.kernel,
    mesh=_mesh,
    out_type=jax.ShapeDtypeStruct((E, EH), f32),
    scratch_types=[
        pltpu.VMEM((NCHUNK, K), jnp.int32),
        pltpu.VMEM((NCHUNK, K), jnp.int32),
        pltpu.VMEM((K, D), f32),
        pltpu.VMEM((K, D), f32),
        pltpu.VMEM((K, D), f32),
        pltpu.VMEM((K, D), f32),
        pltpu.VMEM((K, EH), f32),
        pltpu.VMEM((K, EH), f32),
        pltpu.SemaphoreType.DMA,
        pltpu.SemaphoreType.DMA,
        pltpu.SemaphoreType.DMA,
        pltpu.SemaphoreType.DMA,
        pltpu.SemaphoreType.DMA,
        pltpu.SemaphoreType.DMA,
    ],
)
def _sc_gather_add(pq_hbm, rows_hbm, cols_hbm, s_hbm,
                   idxr, idxc, bufr0, bufc0, bufr1, bufc1, sbuf0, sbuf1,
                   semr0, semc0, semr1, semc1, sems0, sems1):
    wid = lax.axis_index("s") * NC + lax.axis_index("c")
    pltpu.sync_copy(rows_hbm.at[wid], idxr)
    pltpu.sync_copy(cols_hbm.at[wid], idxc)

    slots = ((bufr0, bufc0, sbuf0, semr0, semc0, sems0),
             (bufr1, bufc1, sbuf1, semr1, semc1, sems1))

    def issue(j, s):
        bufr, bufc, _, semr, semc, _ = slots[s]
        pltpu.async_copy(pq_hbm.at[idxr.at[j]], bufr, semr)
        pltpu.async_copy(pq_hbm.at[idxc.at[j]], bufc, semc)

    def process(j, s, first):
        bufr, bufc, sbuf, semr, semc, sems = slots[s]
        pltpu.make_async_copy(pq_hbm.at[idxr.at[j]], bufr, semr).wait()
        pltpu.make_async_copy(pq_hbm.at[idxc.at[j]], bufc, semc).wait()
        if not first:
            pltpu.make_async_copy(
                sbuf, s_hbm.at[pl.ds(wid * EPW, K)], sems).wait()

        def addrow(r, c2):
            for t in range(EH // 16):
                sbuf[r, pl.ds(t * 16, 16)] = (bufr[r, pl.ds(t * 16, 16)]
                                              + bufc[r, pl.ds(EH + t * 16, 16)])
            return c2

        lax.fori_loop(0, K, addrow, 0)
        pltpu.async_copy(sbuf, s_hbm.at[pl.ds(wid * EPW + j * K, K)], sems)

    issue(0, 0)
    issue(1, 1)

    def pair(it, carry):
        j0 = it * 2

        @pl.when(it > 0)
        def _p0():
            process(j0, 0, False)

        @pl.when(it == 0)
        def _p0f():
            process(j0, 0, True)

        @pl.when(j0 + 2 < NCHUNK)
        def _i0():
            issue(j0 + 2, 0)

        @pl.when(it > 0)
        def _p1():
            process(j0 + 1, 1, False)

        @pl.when(it == 0)
        def _p1f():
            process(j0 + 1, 1, True)

        @pl.when(j0 + 3 < NCHUNK)
        def _i1():
            issue(j0 + 3, 1)

        return carry

    lax.fori_loop(0, NCHUNK // 2, pair, 0)
    process(NCHUNK - 1, 0, False)
    pltpu.make_async_copy(sbuf1, s_hbm.at[pl.ds(wid * EPW, K)], sems1).wait()
    pltpu.make_async_copy(sbuf0, s_hbm.at[pl.ds(wid * EPW, K)], sems0).wait()


@functools.partial(
    pl.kernel,
    mesh=_mesh,
    out_type=jax.ShapeDtypeStruct((NC, N, W), f32),
    scratch_types=[
        pltpu.VMEM((K,), jnp.int32),
        pltpu.VMEM((K,), jnp.int32),
        pltpu.VMEM((K, W), f32),
        pltpu.VMEM((K, W), f32),
        pltpu.VMEM_SHARED((NACC, W), f32),
        pltpu.SemaphoreType.DMA,
        pltpu.SemaphoreType.DMA,
        pltpu.SemaphoreType.DMA,
        pltpu.SemaphoreType.DMA,
        pltpu.SemaphoreType.DMA,
        pltpu.SemaphoreType.DMA,
    ],
)
def _sc_scatter_add(m_hbm, rows_hbm, out_hbm, idxc0, idxc1,
                    bufm0, bufm1, acc,
                    semi0, semi1, semm0, semm1, semsc0, semsc1):
    cid = lax.axis_index("c")
    sid = lax.axis_index("s")
    wid = sid * NC + cid
    zero = jnp.zeros((16,), f32)

    def zrow(r, c2):
        for t in range(W // 16):
            bufm0[r, pl.ds(t * 16, 16)] = zero
        return c2

    lax.fori_loop(0, K, zrow, 0)
    for g in range(ZRS // K):
        pltpu.sync_copy(bufm0, acc.at[pl.ds(sid * ZRS + g * K, K)])
    plsc.subcore_barrier()

    slots = ((idxc0, bufm0, semi0, semm0, semsc0),
             (idxc1, bufm1, semi1, semm1, semsc1))

    def issue(j, s):
        idxc, bufm, semi, semm, _ = slots[s]
        pltpu.async_copy(rows_hbm.at[pl.ds(wid * EPW + j * K, K)], idxc, semi)
        pltpu.async_copy(m_hbm.at[pl.ds(wid * EPW + j * K, K)], bufm, semm)

    def start_scatter(j, s):
        idxc, bufm, semi, semm, semsc = slots[s]
        pltpu.make_async_copy(
            rows_hbm.at[pl.ds(wid * EPW + j * K, K)], idxc, semi).wait()
        pltpu.make_async_copy(
            m_hbm.at[pl.ds(wid * EPW + j * K, K)], bufm, semm).wait()
        return pltpu.async_copy(bufm, acc.at[idxc], semsc, add=True)

    issue(0, 0)
    issue(1, 1)

    def pair(it, carry):
        j0 = it * 2
        sc0 = start_scatter(j0, 0)
        sc1 = start_scatter(j0 + 1, 1)
        sc0.wait()

        @pl.when(j0 + 2 < NCHUNK)
        def _i0():
            issue(j0 + 2, 0)

        sc1.wait()

        @pl.when(j0 + 3 < NCHUNK)
        def _i1():
            issue(j0 + 3, 1)

        return carry

    lax.fori_loop(0, NCHUNK // 2, pair, 0)
    start_scatter(NCHUNK - 1, 0).wait()
    plsc.subcore_barrier()

    @pl.when(sid < NS - 1)
    def _copy_full():
        for g in range(ZRS // K):
            pltpu.sync_copy(acc.at[pl.ds(sid * ZRS + g * K, K)],
                            out_hbm.at[cid, pl.ds(sid * ZRS + g * K, K)])

    @pl.when(sid == NS - 1)
    def _copy_tail():
        rem = N - (NS - 1) * ZRS
        for g in range(rem // K):
            pltpu.sync_copy(acc.at[pl.ds((NS - 1) * ZRS + g * K, K)],
                            out_hbm.at[cid, pl.ds((NS - 1) * ZRS + g * K, K)])


def _embed_pq_body(h_ref, we_ref, be_ref, w1_ref, b1_ref, h0_ref, pq_ref):
    h0 = jnp.dot(h_ref[...], we_ref[...], preferred_element_type=f32) + be_ref[...]
    h0_ref[...] = h0
    pq_ref[...] = jnp.dot(h0, w1_ref[...], preferred_element_type=f32) + b1_ref[...]


def _tc_embed_pq(h, we, be, w1, b1):
    return pl.pallas_call(
        _embed_pq_body,
        grid=(N // BN,),
        in_specs=[
            pl.BlockSpec((BN, D), lambda i: (i, 0)),
            pl.BlockSpec((D, D), lambda i: (0, 0)),
            pl.BlockSpec((1, D), lambda i: (0, 0)),
            pl.BlockSpec((D, 2 * EH), lambda i: (0, 0)),
            pl.BlockSpec((1, 2 * EH), lambda i: (0, 0)),
        ],
        out_specs=[
            pl.BlockSpec((BN, D), lambda i: (i, 0)),
            pl.BlockSpec((BN, 2 * EH), lambda i: (i, 0)),
        ],
        out_shape=[
            jax.ShapeDtypeStruct((N, D), f32),
            jax.ShapeDtypeStruct((N, 2 * EH), f32),
        ],
    )(h, we, be, w1, b1)


def _edge_mlp_body(s_ref, w2_ref, b2_ref, m_ref, *, c):
    s = _silu(s_ref[...])
    t = jnp.dot(s, w2_ref[...], preferred_element_type=f32) + b2_ref[...]
    m_ref[...] = jnp.concatenate(
        [_silu(t) * c, jnp.zeros((t.shape[0], W - EH), f32)], axis=1)


def _tc_edge_mlp(s, w2, b2, c):
    return pl.pallas_call(
        functools.partial(_edge_mlp_body, c=c),
        grid=(E // BE,),
        in_specs=[
            pl.BlockSpec((BE, EH), lambda i: (i, 0)),
            pl.BlockSpec((EH, EH), lambda i: (0, 0)),
            pl.BlockSpec((1, EH), lambda i: (0, 0)),
        ],
        out_specs=pl.BlockSpec((BE, W), lambda i: (i, 0)),
        out_shape=jax.ShapeDtypeStruct((E, W), f32),
    )(s, w2, b2)


def _node_body(h_ref, agg_ref, mask_ref, nh_ref, na_ref, nb1_ref, w2_ref,
               nb2_ref, wx_ref, bx_ref, hn_ref, pq_ref):
    h0 = h_ref[...]
    av = agg_ref[...]
    agg = av[0, :, :EH] + av[1, :, :EH]
    t = (jnp.dot(h0, nh_ref[...], preferred_element_type=f32)
         + jnp.dot(agg, na_ref[...], preferred_element_type=f32) + nb1_ref[...])
    t = _silu(t)
    out = jnp.dot(t, w2_ref[...], preferred_element_type=f32) + nb2_ref[...]
    hn = h0 + mask_ref[...] * out
    hn_ref[...] = hn
    pq_ref[...] = jnp.dot(hn, wx_ref[...], preferred_element_type=f32) + bx_ref[...]


def _tc_node(h0, agg, mask, nh, na, nb1, nw2i, nb2, wx, bx):
    return pl.pallas_call(
        _node_body,
        grid=(N // BN,),
        in_specs=[
            pl.BlockSpec((BN, D), lambda i: (i, 0)),
            pl.BlockSpec((NC, BN, W), lambda i: (0, i, 0)),
            pl.BlockSpec((BN, 1), lambda i: (i, 0)),
            pl.BlockSpec((D, D), lambda i: (0, 0)),
            pl.BlockSpec((EH, D), lambda i: (0, 0)),
            pl.BlockSpec((1, D), lambda i: (0, 0)),
            pl.BlockSpec((D, D), lambda i: (0, 0)),
            pl.BlockSpec((1, D), lambda i: (0, 0)),
            pl.BlockSpec((D, 2 * EH), lambda i: (0, 0)),
            pl.BlockSpec((1, 2 * EH), lambda i: (0, 0)),
        ],
        out_specs=[
            pl.BlockSpec((BN, D), lambda i: (i, 0)),
            pl.BlockSpec((BN, 2 * EH), lambda i: (i, 0)),
        ],
        out_shape=[
            jax.ShapeDtypeStruct((N, D), f32),
            jax.ShapeDtypeStruct((N, 2 * EH), f32),
        ],
    )(h0, agg, mask, nh, na, nb1, nw2i, nb2, wx, bx)


def _node_final_body(h_ref, agg_ref, mask_ref, nh_ref, na_ref, nb1_ref,
                     w2_ref, nb2_ref, wo_ref, bo_ref, out_ref):
    h0 = h_ref[...]
    av = agg_ref[...]
    agg = av[0, :, :EH] + av[1, :, :EH]
    t = (jnp.dot(h0, nh_ref[...], preferred_element_type=f32)
         + jnp.dot(agg, na_ref[...], preferred_element_type=f32) + nb1_ref[...])
    t = _silu(t)
    out = jnp.dot(t, w2_ref[...], preferred_element_type=f32) + nb2_ref[...]
    hn = h0 + mask_ref[...] * out
    out_ref[...] = jnp.dot(hn, wo_ref[...], preferred_element_type=f32) + bo_ref[...]


def _tc_node_final(h0, agg, mask, nh, na, nb1, nw2i, nb2, wo, bo):
    return pl.pallas_call(
        _node_final_body,
        grid=(N // BN,),
        in_specs=[
            pl.BlockSpec((BN, D), lambda i: (i, 0)),
            pl.BlockSpec((NC, BN, W), lambda i: (0, i, 0)),
            pl.BlockSpec((BN, 1), lambda i: (i, 0)),
            pl.BlockSpec((D, D), lambda i: (0, 0)),
            pl.BlockSpec((EH, D), lambda i: (0, 0)),
            pl.BlockSpec((1, D), lambda i: (0, 0)),
            pl.BlockSpec((D, D), lambda i: (0, 0)),
            pl.BlockSpec((1, D), lambda i: (0, 0)),
            pl.BlockSpec((D, D), lambda i: (0, 0)),
            pl.BlockSpec((1, D), lambda i: (0, 0)),
        ],
        out_specs=pl.BlockSpec((BN, D), lambda i: (i, 0)),
        out_shape=jax.ShapeDtypeStruct((N, D), f32),
    )(h0, agg, mask, nh, na, nb1, nw2i, nb2, wo, bo)


def kernel(h, edges_a, edges_b, update_mask_a, update_mask_b,
           emb_in_w, emb_in_b, emb_out_w, emb_out_b,
           ew1, eb1, ew2, eb2, nw1, nb1, nw2, nb2):
    rows_a = edges_a[0].reshape(NW, NCHUNK, K)
    cols_a = edges_a[1].reshape(NW, NCHUNK, K)
    rows_b = edges_b[0].reshape(NW, NCHUNK, K)
    cols_b = edges_b[1].reshape(NW, NCHUNK, K)
    w1p = jnp.concatenate([ew1[:, :D, :], ew1[:, D:, :]], axis=-1)
    b1p = jnp.concatenate([eb1, jnp.zeros_like(eb1)], axis=-1)
    nh = nw1[:, :D, :]
    na = nw1[:, D:, :]

    h0, pq = _tc_embed_pq(h, emb_in_w, emb_in_b.reshape(1, D),
                          w1p[0], b1p[0].reshape(1, 2 * EH))
    out = None
    for i in range(L):
        rows, cols = (rows_a, cols_a) if i % 2 == 0 else (rows_b, cols_b)
        mask = update_mask_a if i % 2 == 0 else update_mask_b
        c = 1.0 if i % 2 == 0 else 2.0 / NUM_LATENT
        s = _sc_gather_add(pq, rows, cols)
        m = _tc_edge_mlp(s, ew2[i], eb2[i].reshape(1, EH), c)
        agg = _sc_scatter_add(m, rows.reshape(E))
        if i < L - 1:
            h0, pq = _tc_node(h0, agg, mask, nh[i], na[i],
                              nb1[i].reshape(1, D), nw2[i],
                              nb2[i].reshape(1, D), w1p[i + 1],
                              b1p[i + 1].reshape(1, 2 * EH))
        else:
            out = _tc_node_final(h0, agg, mask, nh[i], na[i],
                                 nb1[i].reshape(1, D), nw2[i],
                                 nb2[i].reshape(1, D), emb_out_w,
                                 emb_out_b.reshape(1, D))
    return out


# speedup vs baseline: 5.0219x; 1.0213x over previous
import functools

import jax
import jax.numpy as jnp
from jax import lax
from jax.experimental import pallas as pl
from jax.experimental.pallas import tpu as pltpu
from jax.experimental.pallas import tpu_sc as plsc

N = 10000
E = 320000
D = 128
EH = 64
L = 4
NUM_LATENT = 1000

NC = 2
NS = 16
NW = NC * NS
EPW = E // NW
K = 80
NCHUNK = EPW // K
NACC = 10240
ZRS = NACC // NS
W = 128

BN = 2000
BE = 4000

f32 = jnp.float32
_mesh = plsc.VectorSubcoreMesh(core_axis_name="c", subcore_axis_name="s")


def _silu(x):
    h = 0.5 * x
    return h + h * jnp.tanh(h)


@functools.partial(
    pl.kernel,
    mesh=_mesh,
    out_type=jax.ShapeDtypeStruct((E, EH), f32),
    scratch_types=[
        pltpu.VMEM((NCHUNK, K), jnp.int32),
        pltpu.VMEM((NCHUNK, K), jnp.int32),
        pltpu.VMEM((K, D), f32),
        pltpu.VMEM((K, D), f32),
        pltpu.VMEM((K, D), f32),
        pltpu.VMEM((K, D), f32),
        pltpu.VMEM((K, EH), f32),
        pltpu.VMEM((K, EH), f32),
        pltpu.SemaphoreType.DMA,
        pltpu.SemaphoreType.DMA,
        pltpu.SemaphoreType.DMA,
        pltpu.SemaphoreType.DMA,
        pltpu.SemaphoreType.DMA,
        pltpu.SemaphoreType.DMA,
    ],
)
def _sc_gather_add(pq_hbm, rows_hbm, cols_hbm, s_hbm,
                   idxr, idxc, bufr0, bufc0, bufr1, bufc1, sbuf0, sbuf1,
                   semr0, semc0, semr1, semc1, sems0, sems1):
    wid = lax.axis_index("s") * NC + lax.axis_index("c")
    pltpu.sync_copy(rows_hbm.at[wid], idxr)
    pltpu.sync_copy(cols_hbm.at[wid], idxc)

    slots = ((bufr0, bufc0, sbuf0, semr0, semc0, sems0),
             (bufr1, bufc1, sbuf1, semr1, semc1, sems1))

    def issue(j, s):
        bufr, bufc, _, semr, semc, _ = slots[s]
        pltpu.async_copy(pq_hbm.at[idxr.at[j]], bufr, semr)
        pltpu.async_copy(pq_hbm.at[idxc.at[j]], bufc, semc)

    def process(j, s, first):
        bufr, bufc, sbuf, semr, semc, sems = slots[s]
        pltpu.make_async_copy(pq_hbm.at[idxr.at[j]], bufr, semr).wait()
        pltpu.make_async_copy(pq_hbm.at[idxc.at[j]], bufc, semc).wait()
        if not first:
            pltpu.make_async_copy(
                sbuf, s_hbm.at[pl.ds(wid * EPW, K)], sems).wait()

        def addrow(r, c2):
            for t in range(EH // 16):
                sbuf[r, pl.ds(t * 16, 16)] = (bufr[r, pl.ds(t * 16, 16)]
                                              + bufc[r, pl.ds(EH + t * 16, 16)])
            return c2

        lax.fori_loop(0, K, addrow, 0)
        pltpu.async_copy(sbuf, s_hbm.at[pl.ds(wid * EPW + j * K, K)], sems)

    issue(0, 0)
    issue(1, 1)

    def pair(it, carry):
        j0 = it * 2

        @pl.when(it > 0)
        def _p0():
            process(j0, 0, False)

        @pl.when(it == 0)
        def _p0f():
            process(j0, 0, True)

        @pl.when(j0 + 2 < NCHUNK)
        def _i0():
            issue(j0 + 2, 0)

        @pl.when(it > 0)
        def _p1():
            process(j0 + 1, 1, False)

        @pl.when(it == 0)
        def _p1f():
            process(j0 + 1, 1, True)

        @pl.when(j0 + 3 < NCHUNK)
        def _i1():
            issue(j0 + 3, 1)

        return carry

    lax.fori_loop(0, NCHUNK // 2, pair, 0)
    process(NCHUNK - 1, 0, False)
    pltpu.make_async_copy(sbuf1, s_hbm.at[pl.ds(wid * EPW, K)], sems1).wait()
    pltpu.make_async_copy(sbuf0, s_hbm.at[pl.ds(wid * EPW, K)], sems0).wait()


@functools.partial(
    pl.kernel,
    mesh=_mesh,
    out_type=jax.ShapeDtypeStruct((NC, N, W), f32),
    scratch_types=[
        pltpu.VMEM((K,), jnp.int32),
        pltpu.VMEM((K,), jnp.int32),
        pltpu.VMEM((K, W), f32),
        pltpu.VMEM((K, W), f32),
        pltpu.VMEM_SHARED((NACC, W), f32),
        pltpu.SemaphoreType.DMA,
        pltpu.SemaphoreType.DMA,
        pltpu.SemaphoreType.DMA,
        pltpu.SemaphoreType.DMA,
        pltpu.SemaphoreType.DMA,
        pltpu.SemaphoreType.DMA,
    ],
)
def _sc_scatter_add(m_hbm, rows_hbm, out_hbm, idxc0, idxc1,
                    bufm0, bufm1, acc,
                    semi0, semi1, semm0, semm1, semsc0, semsc1):
    cid = lax.axis_index("c")
    sid = lax.axis_index("s")
    wid = sid * NC + cid
    zero = jnp.zeros((16,), f32)

    def zrow(r, c2):
        for t in range(W // 16):
            bufm0[r, pl.ds(t * 16, 16)] = zero
        return c2

    lax.fori_loop(0, K, zrow, 0)
    for g in range(ZRS // K):
        pltpu.sync_copy(bufm0, acc.at[pl.ds(sid * ZRS + g * K, K)])
    plsc.subcore_barrier()

    slots = ((idxc0, bufm0, semi0, semm0, semsc0),
             (idxc1, bufm1, semi1, semm1, semsc1))

    def issue(j, s):
        idxc, bufm, semi, semm, _ = slots[s]
        pltpu.async_copy(rows_hbm.at[pl.ds(wid * EPW + j * K, K)], idxc, semi)
        pltpu.async_copy(m_hbm.at[pl.ds(wid * EPW + j * K, K)], bufm, semm)

    def start_scatter(j, s):
        idxc, bufm, semi, semm, semsc = slots[s]
        pltpu.make_async_copy(
            rows_hbm.at[pl.ds(wid * EPW + j * K, K)], idxc, semi).wait()
        pltpu.make_async_copy(
            m_hbm.at[pl.ds(wid * EPW + j * K, K)], bufm, semm).wait()
        return pltpu.async_copy(bufm, acc.at[idxc], semsc, add=True)

    issue(0, 0)
    issue(1, 1)

    def pair(it, carry):
        j0 = it * 2
        sc0 = start_scatter(j0, 0)
        sc1 = start_scatter(j0 + 1, 1)
        sc0.wait()

        @pl.when(j0 + 2 < NCHUNK)
        def _i0():
            issue(j0 + 2, 0)

        sc1.wait()

        @pl.when(j0 + 3 < NCHUNK)
        def _i1():
            issue(j0 + 3, 1)

        return carry

    lax.fori_loop(0, NCHUNK // 2, pair, 0)
    start_scatter(NCHUNK - 1, 0).wait()
    plsc.subcore_barrier()

    @pl.when(sid < NS - 1)
    def _copy_full():
        for g in range(ZRS // K):
            pltpu.sync_copy(acc.at[pl.ds(sid * ZRS + g * K, K)],
                            out_hbm.at[cid, pl.ds(sid * ZRS + g * K, K)])

    @pl.when(sid == NS - 1)
    def _copy_tail():
        rem = N - (NS - 1) * ZRS
        for g in range(rem // K):
            pltpu.sync_copy(acc.at[pl.ds((NS - 1) * ZRS + g * K, K)],
                            out_hbm.at[cid, pl.ds((NS - 1) * ZRS + g * K, K)])


def _embed_pq_body(h_ref, we_ref, be_ref, w1_ref, b1_ref, h0_ref, pq_ref):
    h0 = jnp.dot(h_ref[...], we_ref[...], preferred_element_type=f32) + be_ref[...]
    h0_ref[...] = h0
    pq_ref[...] = jnp.dot(h0, w1_ref[...], preferred_element_type=f32) + b1_ref[...]


def _tc_embed_pq(h, we, be, w1, b1):
    return pl.pallas_call(
        _embed_pq_body,
        grid=(N // BN,),
        in_specs=[
            pl.BlockSpec((BN, D), lambda i: (i, 0)),
            pl.BlockSpec((D, D), lambda i: (0, 0)),
            pl.BlockSpec((1, D), lambda i: (0, 0)),
            pl.BlockSpec((D, 2 * EH), lambda i: (0, 0)),
            pl.BlockSpec((1, 2 * EH), lambda i: (0, 0)),
        ],
        out_specs=[
            pl.BlockSpec((BN, D), lambda i: (i, 0)),
            pl.BlockSpec((BN, 2 * EH), lambda i: (i, 0)),
        ],
        out_shape=[
            jax.ShapeDtypeStruct((N, D), f32),
            jax.ShapeDtypeStruct((N, 2 * EH), f32),
        ],
    )(h, we, be, w1, b1)


def _edge_mlp_body(s_ref, w2_ref, b2_ref, m_ref, *, c):
    s = _silu(s_ref[...])
    t = jnp.dot(s, w2_ref[...], preferred_element_type=f32) + b2_ref[...]
    m_ref[...] = jnp.concatenate(
        [_silu(t) * c, jnp.zeros((t.shape[0], W - EH), f32)], axis=1)


def _tc_edge_mlp(s, w2, b2, c):
    return pl.pallas_call(
        functools.partial(_edge_mlp_body, c=c),
        grid=(E // BE,),
        in_specs=[
            pl.BlockSpec((BE, EH), lambda i: (i, 0)),
            pl.BlockSpec((EH, EH), lambda i: (0, 0)),
            pl.BlockSpec((1, EH), lambda i: (0, 0)),
        ],
        out_specs=pl.BlockSpec((BE, W), lambda i: (i, 0)),
        out_shape=jax.ShapeDtypeStruct((E, W), f32),
    )(s, w2, b2)


def _node_body(h_ref, agg_ref, mask_ref, nh_ref, na_ref, nb1_ref, w2_ref,
               nb2_ref, wx_ref, bx_ref, hn_ref, pq_ref):
    h0 = h_ref[...]
    av = agg_ref[...]
    agg = av[0, :, :EH] + av[1, :, :EH]
    t = (jnp.dot(h0, nh_ref[...], preferred_element_type=f32)
         + jnp.dot(agg, na_ref[...], preferred_element_type=f32) + nb1_ref[...])
    t = _silu(t)
    out = jnp.dot(t, w2_ref[...], preferred_element_type=f32) + nb2_ref[...]
    hn = h0 + mask_ref[...] * out
    hn_ref[...] = hn
    pq_ref[...] = jnp.dot(hn, wx_ref[...], preferred_element_type=f32) + bx_ref[...]


def _tc_node(h0, agg, mask, nh, na, nb1, nw2i, nb2, wx, bx):
    return pl.pallas_call(
        _node_body,
        grid=(N // BN,),
        in_specs=[
            pl.BlockSpec((BN, D), lambda i: (i, 0)),
            pl.BlockSpec((NC, BN, W), lambda i: (0, i, 0)),
            pl.BlockSpec((BN, 1), lambda i: (i, 0)),
            pl.BlockSpec((D, D), lambda i: (0, 0)),
            pl.BlockSpec((EH, D), lambda i: (0, 0)),
            pl.BlockSpec((1, D), lambda i: (0, 0)),
            pl.BlockSpec((D, D), lambda i: (0, 0)),
            pl.BlockSpec((1, D), lambda i: (0, 0)),
            pl.BlockSpec((D, 2 * EH), lambda i: (0, 0)),
            pl.BlockSpec((1, 2 * EH), lambda i: (0, 0)),
        ],
        out_specs=[
            pl.BlockSpec((BN, D), lambda i: (i, 0)),
            pl.BlockSpec((BN, 2 * EH), lambda i: (i, 0)),
        ],
        out_shape=[
            jax.ShapeDtypeStruct((N, D), f32),
            jax.ShapeDtypeStruct((N, 2 * EH), f32),
        ],
    )(h0, agg, mask, nh, na, nb1, nw2i, nb2, wx, bx)


def _node_final_body(h_ref, agg_ref, mask_ref, nh_ref, na_ref, nb1_ref,
                     w2_ref, nb2_ref, wo_ref, bo_ref, out_ref):
    h0 = h_ref[...]
    av = agg_ref[...]
    agg = av[0, :, :EH] + av[1, :, :EH]
    t = (jnp.dot(h0, nh_ref[...], preferred_element_type=f32)
         + jnp.dot(agg, na_ref[...], preferred_element_type=f32) + nb1_ref[...])
    t = _silu(t)
    out = jnp.dot(t, w2_ref[...], preferred_element_type=f32) + nb2_ref[...]
    hn = h0 + mask_ref[...] * out
    out_ref[...] = jnp.dot(hn, wo_ref[...], preferred_element_type=f32) + bo_ref[...]


def _tc_node_final(h0, agg, mask, nh, na, nb1, nw2i, nb2, wo, bo):
    return pl.pallas_call(
        _node_final_body,
        grid=(N // BN,),
        in_specs=[
            pl.BlockSpec((BN, D), lambda i: (i, 0)),
            pl.BlockSpec((NC, BN, W), lambda i: (0, i, 0)),
            pl.BlockSpec((BN, 1), lambda i: (i, 0)),
            pl.BlockSpec((D, D), lambda i: (0, 0)),
            pl.BlockSpec((EH, D), lambda i: (0, 0)),
            pl.BlockSpec((1, D), lambda i: (0, 0)),
            pl.BlockSpec((D, D), lambda i: (0, 0)),
            pl.BlockSpec((1, D), lambda i: (0, 0)),
            pl.BlockSpec((D, D), lambda i: (0, 0)),
            pl.BlockSpec((1, D), lambda i: (0, 0)),
        ],
        out_specs=pl.BlockSpec((BN, D), lambda i: (i, 0)),
        out_shape=jax.ShapeDtypeStruct((N, D), f32),
    )(h0, agg, mask, nh, na, nb1, nw2i, nb2, wo, bo)


def kernel(h, edges_a, edges_b, update_mask_a, update_mask_b,
           emb_in_w, emb_in_b, emb_out_w, emb_out_b,
           ew1, eb1, ew2, eb2, nw1, nb1, nw2, nb2):
    rows_a = edges_a[0].reshape(NW, NCHUNK, K)
    cols_a = edges_a[1].reshape(NW, NCHUNK, K)
    rows_b = edges_b[0].reshape(NW, NCHUNK, K)
    cols_b = edges_b[1].reshape(NW, NCHUNK, K)
    w1p = jnp.concatenate([ew1[:, :D, :], ew1[:, D:, :]], axis=-1)
    b1p = jnp.concatenate([eb1, jnp.zeros_like(eb1)], axis=-1)
    nh = nw1[:, :D, :]
    na = nw1[:, D:, :]

    h0, pq = _tc_embed_pq(h, emb_in_w, emb_in_b.reshape(1, D),
                          w1p[0], b1p[0].reshape(1, 2 * EH))
    out = None
    for i in range(L):
        rows, cols = (rows_a, cols_a) if i % 2 == 0 else (rows_b, cols_b)
        mask = update_mask_a if i % 2 == 0 else update_mask_b
        c = 1.0 if i % 2 == 0 else 2.0 / NUM_LATENT
        s = _sc_gather_add(pq, rows, cols)
        m = _tc_edge_mlp(s, ew2[i], eb2[i].reshape(1, EH), c)
        agg = _sc_scatter_add(m, rows.reshape(E))
        if i < L - 1:
            h0, pq = _tc_node(h0, agg, mask, nh[i], na[i],
                              nb1[i].reshape(1, D), nw2[i],
                              nb2[i].reshape(1, D), w1p[i + 1],
                              b1p[i + 1].reshape(1, 2 * EH))
        else:
            out = _tc_node_final(h0, agg, mask, nh[i], na[i],
                                 nb1[i].reshape(1, D), nw2[i],
                                 nb2[i].reshape(1, D), emb_out_w,
                                 emb_out_b.reshape(1, D))
    return out


# speedup vs baseline: 5.1468x; 1.0249x over previous
import functools

import jax
import jax.numpy as jnp
from jax import lax
from jax.experimental import pallas as pl
from jax.experimental.pallas import tpu as pltpu
from jax.experimental.pallas import tpu_sc as plsc

N = 10000
E = 320000
D = 128
EH = 64
L = 4
NUM_LATENT = 1000

NC = 2
NS = 16
NW = NC * NS
K = 80
NACC = 10240
ZRS = NACC // NS
W = 128

E1 = 204800
E2 = E - E1

BN = 2000
BE = 1600

f32 = jnp.float32
_mesh = plsc.VectorSubcoreMesh(core_axis_name="c", subcore_axis_name="s")


def _silu(x):
    h = 0.5 * x
    return h + h * jnp.tanh(h)


def _make_gather(ne):
    epw = ne // NW
    nchunk = epw // K

    @functools.partial(
        pl.kernel,
        mesh=_mesh,
        out_type=jax.ShapeDtypeStruct((ne, EH), f32),
        scratch_types=[
            pltpu.VMEM((nchunk, K), jnp.int32),
            pltpu.VMEM((nchunk, K), jnp.int32),
            pltpu.VMEM((K, D), f32),
            pltpu.VMEM((K, D), f32),
            pltpu.VMEM((K, D), f32),
            pltpu.VMEM((K, D), f32),
            pltpu.VMEM((K, EH), f32),
            pltpu.VMEM((K, EH), f32),
            pltpu.SemaphoreType.DMA,
            pltpu.SemaphoreType.DMA,
            pltpu.SemaphoreType.DMA,
            pltpu.SemaphoreType.DMA,
            pltpu.SemaphoreType.DMA,
            pltpu.SemaphoreType.DMA,
        ],
    )
    def gather(pq_hbm, rows_hbm, cols_hbm, s_hbm,
               idxr, idxc, bufr0, bufc0, bufr1, bufc1, sbuf0, sbuf1,
               semr0, semc0, semr1, semc1, sems0, sems1):
        wid = lax.axis_index("s") * NC + lax.axis_index("c")
        pltpu.sync_copy(rows_hbm.at[wid], idxr)
        pltpu.sync_copy(cols_hbm.at[wid], idxc)

        slots = ((bufr0, bufc0, sbuf0, semr0, semc0, sems0),
                 (bufr1, bufc1, sbuf1, semr1, semc1, sems1))

        def issue(j, s):
            bufr, bufc, _, semr, semc, _ = slots[s]
            pltpu.async_copy(pq_hbm.at[idxr.at[j]], bufr, semr)
            pltpu.async_copy(pq_hbm.at[idxc.at[j]], bufc, semc)

        def process(j, s, first):
            bufr, bufc, sbuf, semr, semc, sems = slots[s]
            pltpu.make_async_copy(pq_hbm.at[idxr.at[j]], bufr, semr).wait()
            pltpu.make_async_copy(pq_hbm.at[idxc.at[j]], bufc, semc).wait()
            if not first:
                pltpu.make_async_copy(
                    sbuf, s_hbm.at[pl.ds(wid * epw, K)], sems).wait()

            def addrow(r, c2):
                for t in range(EH // 16):
                    sbuf[r, pl.ds(t * 16, 16)] = (
                        bufr[r, pl.ds(t * 16, 16)]
                        + bufc[r, pl.ds(EH + t * 16, 16)])
                return c2

            lax.fori_loop(0, K, addrow, 0)
            pltpu.async_copy(sbuf, s_hbm.at[pl.ds(wid * epw + j * K, K)], sems)

        issue(0, 0)
        issue(1, 1)

        def pair(it, carry):
            j0 = it * 2

            @pl.when(it > 0)
            def _p0():
                process(j0, 0, False)

            @pl.when(it == 0)
            def _p0f():
                process(j0, 0, True)

            @pl.when(j0 + 2 < nchunk)
            def _i0():
                issue(j0 + 2, 0)

            @pl.when(it > 0)
            def _p1():
                process(j0 + 1, 1, False)

            @pl.when(it == 0)
            def _p1f():
                process(j0 + 1, 1, True)

            @pl.when(j0 + 3 < nchunk)
            def _i1():
                issue(j0 + 3, 1)

            return carry

        lax.fori_loop(0, nchunk // 2, pair, 0)
        if nchunk % 2:
            process(nchunk - 1, 0, False)
        pltpu.make_async_copy(sbuf1, s_hbm.at[pl.ds(wid * epw, K)], sems1).wait()
        pltpu.make_async_copy(sbuf0, s_hbm.at[pl.ds(wid * epw, K)], sems0).wait()

    return gather


def _make_scatter(ne):
    epw = ne // NW
    nchunk = epw // K

    @functools.partial(
        pl.kernel,
        mesh=_mesh,
        out_type=jax.ShapeDtypeStruct((NC, N, W), f32),
        scratch_types=[
            pltpu.VMEM((K,), jnp.int32),
            pltpu.VMEM((K,), jnp.int32),
            pltpu.VMEM((K, W), f32),
            pltpu.VMEM((K, W), f32),
            pltpu.VMEM_SHARED((NACC, W), f32),
            pltpu.SemaphoreType.DMA,
            pltpu.SemaphoreType.DMA,
            pltpu.SemaphoreType.DMA,
            pltpu.SemaphoreType.DMA,
            pltpu.SemaphoreType.DMA,
            pltpu.SemaphoreType.DMA,
        ],
    )
    def scatter(m_hbm, rows_hbm, out_hbm, idxc0, idxc1, bufm0, bufm1, acc,
                semi0, semi1, semm0, semm1, semsc0, semsc1):
        cid = lax.axis_index("c")
        sid = lax.axis_index("s")
        wid = sid * NC + cid
        zero = jnp.zeros((16,), f32)

        def zrow(r, c2):
            for t in range(W // 16):
                bufm0[r, pl.ds(t * 16, 16)] = zero
            return c2

        lax.fori_loop(0, K, zrow, 0)
        for g in range(ZRS // K):
            pltpu.sync_copy(bufm0, acc.at[pl.ds(sid * ZRS + g * K, K)])
        plsc.subcore_barrier()

        slots = ((idxc0, bufm0, semi0, semm0, semsc0),
                 (idxc1, bufm1, semi1, semm1, semsc1))

        def issue(j, s):
            idxc, bufm, semi, semm, _ = slots[s]
            pltpu.async_copy(rows_hbm.at[pl.ds(wid * epw + j * K, K)],
                             idxc, semi)
            pltpu.async_copy(m_hbm.at[pl.ds(wid * epw + j * K, K)],
                             bufm, semm)

        def start_scatter(j, s):
            idxc, bufm, semi, semm, semsc = slots[s]
            pltpu.make_async_copy(
                rows_hbm.at[pl.ds(wid * epw + j * K, K)], idxc, semi).wait()
            pltpu.make_async_copy(
                m_hbm.at[pl.ds(wid * epw + j * K, K)], bufm, semm).wait()
            return pltpu.async_copy(bufm, acc.at[idxc], semsc, add=True)

        issue(0, 0)
        issue(1, 1)

        def pair(it, carry):
            j0 = it * 2
            sc0 = start_scatter(j0, 0)
            sc1 = start_scatter(j0 + 1, 1)
            sc0.wait()

            @pl.when(j0 + 2 < nchunk)
            def _i0():
                issue(j0 + 2, 0)

            sc1.wait()

            @pl.when(j0 + 3 < nchunk)
            def _i1():
                issue(j0 + 3, 1)

            return carry

        lax.fori_loop(0, nchunk // 2, pair, 0)
        if nchunk % 2:
            start_scatter(nchunk - 1, 0).wait()
        plsc.subcore_barrier()

        @pl.when(sid < NS - 1)
        def _copy_full():
            for g in range(ZRS // K):
                pltpu.sync_copy(acc.at[pl.ds(sid * ZRS + g * K, K)],
                                out_hbm.at[cid, pl.ds(sid * ZRS + g * K, K)])

        @pl.when(sid == NS - 1)
        def _copy_tail():
            rem = N - (NS - 1) * ZRS
            for g in range(rem // K):
                pltpu.sync_copy(
                    acc.at[pl.ds((NS - 1) * ZRS + g * K, K)],
                    out_hbm.at[cid, pl.ds((NS - 1) * ZRS + g * K, K)])

    return scatter


_gather1 = _make_gather(E1)
_gather2 = _make_gather(E2)
_scatter1 = _make_scatter(E1)
_scatter2 = _make_scatter(E2)


def _embed_pq_body(h_ref, we_ref, be_ref, w1_ref, b1_ref, h0_ref, pq_ref):
    h0 = jnp.dot(h_ref[...], we_ref[...], preferred_element_type=f32) + be_ref[...]
    h0_ref[...] = h0
    pq_ref[...] = jnp.dot(h0, w1_ref[...], preferred_element_type=f32) + b1_ref[...]


def _tc_embed_pq(h, we, be, w1, b1):
    return pl.pallas_call(
        _embed_pq_body,
        grid=(N // BN,),
        in_specs=[
            pl.BlockSpec((BN, D), lambda i: (i, 0)),
            pl.BlockSpec((D, D), lambda i: (0, 0)),
            pl.BlockSpec((1, D), lambda i: (0, 0)),
            pl.BlockSpec((D, 2 * EH), lambda i: (0, 0)),
            pl.BlockSpec((1, 2 * EH), lambda i: (0, 0)),
        ],
        out_specs=[
            pl.BlockSpec((BN, D), lambda i: (i, 0)),
            pl.BlockSpec((BN, 2 * EH), lambda i: (i, 0)),
        ],
        out_shape=[
            jax.ShapeDtypeStruct((N, D), f32),
            jax.ShapeDtypeStruct((N, 2 * EH), f32),
        ],
    )(h, we, be, w1, b1)


def _edge_mlp_body(s_ref, w2_ref, b2_ref, m_ref, *, c):
    s = _silu(s_ref[...])
    t = jnp.dot(s, w2_ref[...], preferred_element_type=f32) + b2_ref[...]
    m_ref[...] = jnp.concatenate(
        [_silu(t) * c, jnp.zeros((t.shape[0], W - EH), f32)], axis=1)


def _tc_edge_mlp(s, w2, b2, c):
    ne = s.shape[0]
    return pl.pallas_call(
        functools.partial(_edge_mlp_body, c=c),
        grid=(ne // BE,),
        in_specs=[
            pl.BlockSpec((BE, EH), lambda i: (i, 0)),
            pl.BlockSpec((EH, EH), lambda i: (0, 0)),
            pl.BlockSpec((1, EH), lambda i: (0, 0)),
        ],
        out_specs=pl.BlockSpec((BE, W), lambda i: (i, 0)),
        out_shape=jax.ShapeDtypeStruct((ne, W), f32),
    )(s, w2, b2)


def _node_core(h0, a1, a2, nh_ref, na_ref, nb1_ref, w2_ref, nb2_ref, mask):
    agg = (a1[0, :, :EH] + a1[1, :, :EH]) + (a2[0, :, :EH] + a2[1, :, :EH])
    t = (jnp.dot(h0, nh_ref[...], preferred_element_type=f32)
         + jnp.dot(agg, na_ref[...], preferred_element_type=f32) + nb1_ref[...])
    t = _silu(t)
    out = jnp.dot(t, w2_ref[...], preferred_element_type=f32) + nb2_ref[...]
    return h0 + mask * out


def _node_body(h_ref, a1_ref, a2_ref, mask_ref, nh_ref, na_ref, nb1_ref,
               w2_ref, nb2_ref, wx_ref, bx_ref, hn_ref, pq_ref):
    hn = _node_core(h_ref[...], a1_ref[...], a2_ref[...], nh_ref, na_ref,
                    nb1_ref, w2_ref, nb2_ref, mask_ref[...])
    hn_ref[...] = hn
    pq_ref[...] = jnp.dot(hn, wx_ref[...], preferred_element_type=f32) + bx_ref[...]


_NODE_IN_SPECS = [
    pl.BlockSpec((BN, D), lambda i: (i, 0)),
    pl.BlockSpec((NC, BN, W), lambda i: (0, i, 0)),
    pl.BlockSpec((NC, BN, W), lambda i: (0, i, 0)),
    pl.BlockSpec((BN, 1), lambda i: (i, 0)),
    pl.BlockSpec((D, D), lambda i: (0, 0)),
    pl.BlockSpec((EH, D), lambda i: (0, 0)),
    pl.BlockSpec((1, D), lambda i: (0, 0)),
    pl.BlockSpec((D, D), lambda i: (0, 0)),
    pl.BlockSpec((1, D), lambda i: (0, 0)),
]


def _tc_node(h0, a1, a2, mask, nh, na, nb1, nw2i, nb2, wx, bx):
    return pl.pallas_call(
        _node_body,
        grid=(N // BN,),
        in_specs=_NODE_IN_SPECS + [
            pl.BlockSpec((D, 2 * EH), lambda i: (0, 0)),
            pl.BlockSpec((1, 2 * EH), lambda i: (0, 0)),
        ],
        out_specs=[
            pl.BlockSpec((BN, D), lambda i: (i, 0)),
            pl.BlockSpec((BN, 2 * EH), lambda i: (i, 0)),
        ],
        out_shape=[
            jax.ShapeDtypeStruct((N, D), f32),
            jax.ShapeDtypeStruct((N, 2 * EH), f32),
        ],
    )(h0, a1, a2, mask, nh, na, nb1, nw2i, nb2, wx, bx)


def _node_final_body(h_ref, a1_ref, a2_ref, mask_ref, nh_ref, na_ref,
                     nb1_ref, w2_ref, nb2_ref, wo_ref, bo_ref, out_ref):
    hn = _node_core(h_ref[...], a1_ref[...], a2_ref[...], nh_ref, na_ref,
                    nb1_ref, w2_ref, nb2_ref, mask_ref[...])
    out_ref[...] = jnp.dot(hn, wo_ref[...], preferred_element_type=f32) + bo_ref[...]


def _tc_node_final(h0, a1, a2, mask, nh, na, nb1, nw2i, nb2, wo, bo):
    return pl.pallas_call(
        _node_final_body,
        grid=(N // BN,),
        in_specs=_NODE_IN_SPECS + [
            pl.BlockSpec((D, D), lambda i: (0, 0)),
            pl.BlockSpec((1, D), lambda i: (0, 0)),
        ],
        out_specs=pl.BlockSpec((BN, D), lambda i: (i, 0)),
        out_shape=jax.ShapeDtypeStruct((N, D), f32),
    )(h0, a1, a2, mask, nh, na, nb1, nw2i, nb2, wo, bo)


def kernel(h, edges_a, edges_b, update_mask_a, update_mask_b,
           emb_in_w, emb_in_b, emb_out_w, emb_out_b,
           ew1, eb1, ew2, eb2, nw1, nb1, nw2, nb2):
    def split(e):
        r, c = e[0], e[1]
        return ((r[:E1].reshape(NW, -1, K), c[:E1].reshape(NW, -1, K), r[:E1]),
                (r[E1:].reshape(NW, -1, K), c[E1:].reshape(NW, -1, K), r[E1:]))

    ga, gb = split(edges_a), split(edges_b)
    w1p = jnp.concatenate([ew1[:, :D, :], ew1[:, D:, :]], axis=-1)
    b1p = jnp.concatenate([eb1, jnp.zeros_like(eb1)], axis=-1)
    nh = nw1[:, :D, :]
    na = nw1[:, D:, :]

    h0, pq = _tc_embed_pq(h, emb_in_w, emb_in_b.reshape(1, D),
                          w1p[0], b1p[0].reshape(1, 2 * EH))
    out = None
    for i in range(L):
        (r1, c1, f1), (r2, c2, f2) = ga if i % 2 == 0 else gb
        mask = update_mask_a if i % 2 == 0 else update_mask_b
        c = 1.0 if i % 2 == 0 else 2.0 / NUM_LATENT
        eb2i = eb2[i].reshape(1, EH)
        s1 = _gather1(pq, r1, c1)
        s2 = _gather2(pq, r2, c2)
        m1 = _tc_edge_mlp(s1, ew2[i], eb2i, c)
        m2 = _tc_edge_mlp(s2, ew2[i], eb2i, c)
        a1 = _scatter1(m1, f1)
        a2 = _scatter2(m2, f2)
        if i < L - 1:
            h0, pq = _tc_node(h0, a1, a2, mask, nh[i], na[i],
                              nb1[i].reshape(1, D), nw2[i],
                              nb2[i].reshape(1, D), w1p[i + 1],
                              b1p[i + 1].reshape(1, 2 * EH))
        else:
            out = _tc_node_final(h0, a1, a2, mask, nh[i], na[i],
                                 nb1[i].reshape(1, D), nw2[i],
                                 nb2[i].reshape(1, D), emb_out_w,
                                 emb_out_b.reshape(1, D))
    return out
